# baseline (speedup 1.0000x reference)
import functools

import jax
import jax.numpy as jnp
from jax import lax
from jax.experimental import pallas as pl
from jax.experimental.pallas import tpu as pltpu
from jax.experimental.pallas import tpu_sc as plsc

N = 50000
E = 800000
D = 64

NC = 2
NS = 16
HALF = N // NC
TRASH = 16
ACC = HALF + TRASH

CHUNK = 128
NB = 8
BLKE = NB * CHUNK
ET = 50176
E_PAD = NS * ET
EROWS = E_PAD // CHUNK
TROWS = ET // CHUNK
NBLKP = TROWS // NB

TCAP = ET + BLKE
NRMAX = TCAP // CHUNK
NBMAX = TCAP // BLKE

NZFULL = 195
ZREM = ACC - NZFULL * CHUNK
RREM = HALF - NZFULL * CHUNK
NZITER = NZFULL // NS + 1

ZCH = 1000
NZC = HALF // ZCH

_mesh = plsc.VectorSubcoreMesh(core_axis_name="c", subcore_axis_name="s")
_sc_params = pltpu.CompilerParams(use_tc_tiling_on_sc=False)
_sc_params_nl = pltpu.CompilerParams(use_tc_tiling_on_sc=False,
                                     needs_layout_passes=False)


@functools.partial(
    pl.kernel,
    out_type=(jax.ShapeDtypeStruct((NC, NS, TCAP), jnp.int32),
              jax.ShapeDtypeStruct((NC, NS, TCAP), jnp.int32),
              jax.ShapeDtypeStruct((NC, NS, 16), jnp.int32),
              jax.ShapeDtypeStruct((N,), jnp.float32),
              jax.ShapeDtypeStruct((N,), jnp.float32)),
    mesh=_mesh,
    compiler_params=_sc_params_nl,
    scratch_types=[
        pltpu.VMEM((2, NB, CHUNK), jnp.int32),
        pltpu.VMEM((2, NB, CHUNK), jnp.int32),
        pltpu.VMEM((TCAP,), jnp.int32),
        pltpu.VMEM((TCAP,), jnp.int32),
        pltpu.VMEM((CHUNK,), jnp.int32),
        pltpu.VMEM((CHUNK,), jnp.int32),
        pltpu.VMEM((CHUNK,), jnp.float32),
        pltpu.VMEM((CHUNK,), jnp.float32),
        pltpu.VMEM((16,), jnp.int32),
        pltpu.VMEM_SHARED((ACC,), jnp.float32),
        pltpu.VMEM_SHARED((ACC,), jnp.float32),
        pltpu.SemaphoreType.DMA,
    ],
)
def _prep_kernel(src_hbm, dst_hbm, zeros1_hbm,
                 csrc_hbm, cdst_hbm, cnt_hbm, din_hbm, dout_hbm,
                 srcb, dstb, csrcv, cdstv, dlocb, slocb, ones, zb, cntv,
                 accin, accout, semi):
    c = lax.axis_index("c")
    s = lax.axis_index("s")
    off = c * HALF
    lane = lax.iota(jnp.int32, 16)

    for j in range(CHUNK // 16):
        ones[pl.ds(j * 16, 16)] = jnp.full((16,), 1.0, jnp.float32)
    pltpu.sync_copy(zeros1_hbm, zb)

    def zbody(j, carry):
        i = s + j * NS
        @pl.when(i < NZFULL)
        def _():
            pltpu.sync_copy(zb, accin.at[pl.ds(i * CHUNK, CHUNK)])
            pltpu.sync_copy(zb, accout.at[pl.ds(i * CHUNK, CHUNK)])
        return carry

    lax.fori_loop(0, NZITER, zbody, 0)
    @pl.when(s == NS - 1)
    def _():
        pltpu.sync_copy(zb.at[pl.ds(0, ZREM)], accin.at[pl.ds(NZFULL * CHUNK, ZREM)])
        pltpu.sync_copy(zb.at[pl.ds(0, ZREM)], accout.at[pl.ds(NZFULL * CHUNK, ZREM)])
    plsc.subcore_barrier()

    base0 = s * TROWS
    pltpu.async_copy(src_hbm.at[pl.ds(base0, NB)], srcb.at[0], semi)
    pltpu.async_copy(dst_hbm.at[pl.ds(base0, NB)], dstb.at[0], semi)

    def blk_body(blk, pos):
        cur = lax.rem(blk, 2)
        nxt = 1 - cur
        base = s * TROWS + blk * NB
        pltpu.make_async_copy(src_hbm.at[pl.ds(base, NB)], srcb.at[cur], semi).wait()
        pltpu.make_async_copy(dst_hbm.at[pl.ds(base, NB)], dstb.at[cur], semi).wait()

        @pl.when(blk + 1 < NBLKP)
        def _():
            nb_ = base + NB
            pltpu.async_copy(src_hbm.at[pl.ds(nb_, NB)], srcb.at[nxt], semi)
            pltpu.async_copy(dst_hbm.at[pl.ds(nb_, NB)], dstb.at[nxt], semi)

        one16 = jnp.full((16,), 1, jnp.int32)
        zero16 = jnp.zeros((16,), jnp.int32)

        def row_body(r, posv16):
            for j in range(CHUNK // 16):
                sv = srcb[cur, r, pl.ds(j * 16, 16)]
                dv = dstb[cur, r, pl.ds(j * 16, 16)]
                dl = dv - off
                dm = (dl >= 0) & (dl < HALF)
                dlocb[pl.ds(j * 16, 16)] = jnp.where(dm, dl, HALF + lane)
                sl = sv - off
                sm = (sl >= 0) & (sl < HALF)
                slocb[pl.ds(j * 16, 16)] = jnp.where(sm, sl, HALF + lane)
                dmi = jnp.where(dm, one16, zero16)
                inc = plsc.cumsum(dmi)
                idxv = posv16 + (inc - dmi)
                plsc.store_scatter(csrcv, [idxv], sv, mask=dm)
                plsc.store_scatter(cdstv, [idxv], dl, mask=dm)
                posv16 = posv16 + plsc.all_reduce_population_count(dm)
            pltpu.sync_copy(ones, accin.at[dlocb], add=True)
            pltpu.sync_copy(ones, accout.at[slocb], add=True)
            return posv16

        return lax.fori_loop(0, NB, row_body, pos)

    posv = lax.fori_loop(0, NBLKP, blk_body, jnp.zeros((16,), jnp.int32))

    lane = lax.iota(jnp.int32, 16)
    zero16 = jnp.zeros((16,), jnp.int32)
    for i in range(BLKE // 16):
        idxv = posv + (i * 16) + lane
        plsc.store_scatter(csrcv, [idxv], zero16)
        plsc.store_scatter(cdstv, [idxv], HALF + lane)
    cntv[pl.ds(0, 16)] = jnp.right_shift(posv + (BLKE - 1), 10)
    pltpu.sync_copy(cntv, cnt_hbm.at[c, s])
    pltpu.sync_copy(csrcv, csrc_hbm.at[c, s])
    pltpu.sync_copy(cdstv, cdst_hbm.at[c, s])
    plsc.subcore_barrier()

    def rbody(j, carry):
        i = s + j * NS
        @pl.when(i < NZFULL)
        def _():
            pltpu.sync_copy(accin.at[pl.ds(i * CHUNK, CHUNK)], zb)
            pltpu.sync_copy(zb, din_hbm.at[pl.ds(c * HALF + i * CHUNK, CHUNK)])
            pltpu.sync_copy(accout.at[pl.ds(i * CHUNK, CHUNK)], zb)
            pltpu.sync_copy(zb, dout_hbm.at[pl.ds(c * HALF + i * CHUNK, CHUNK)])
        return carry

    lax.fori_loop(0, NZITER, rbody, 0)
    @pl.when(s == NS - 1)
    def _():
        pltpu.sync_copy(accin.at[pl.ds(NZFULL * CHUNK, RREM)], zb.at[pl.ds(0, RREM)])
        pltpu.sync_copy(zb.at[pl.ds(0, RREM)],
                        din_hbm.at[pl.ds(c * HALF + NZFULL * CHUNK, RREM)])
        pltpu.sync_copy(accout.at[pl.ds(NZFULL * CHUNK, RREM)], zb.at[pl.ds(0, RREM)])
        pltpu.sync_copy(zb.at[pl.ds(0, RREM)],
                        dout_hbm.at[pl.ds(c * HALF + NZFULL * CHUNK, RREM)])


@functools.partial(
    pl.kernel,
    out_type=jax.ShapeDtypeStruct((N, D), jnp.float32),
    mesh=_mesh,
    compiler_params=_sc_params_nl,
    scratch_types=[
        pltpu.VMEM((2, NB, CHUNK), jnp.int32),
        pltpu.VMEM((2, NB, CHUNK), jnp.int32),
        pltpu.VMEM((3, CHUNK, D), jnp.float32),
        pltpu.VMEM((16,), jnp.int32),
        pltpu.VMEM_SHARED((ACC, D), jnp.float32),
        pltpu.SemaphoreType.DMA,
        pltpu.SemaphoreType.DMA,
        pltpu.SemaphoreType.DMA,
    ],
)
def _agg_kernel(csrc_hbm, cdst_hbm, cnt_hbm, h_hbm, zeros_hbm, out_hbm,
                srcb, dstb, rows3, cntv, acc, semi, semg, sems):
    zrows = rows3.at[0]
    c = lax.axis_index("c")
    s = lax.axis_index("s")

    pltpu.sync_copy(cnt_hbm.at[c, s], cntv)
    nblk = jnp.max(cntv[pl.ds(0, 16)])

    def zbody(j, carry):
        i = s + j * NS
        @pl.when(i < NZC)
        def _():
            pltpu.sync_copy(zeros_hbm, acc.at[pl.ds(i * ZCH, ZCH)])
        return carry

    lax.fori_loop(0, NZC // NS + 1, zbody, 0)
    @pl.when(s == NS - 1)
    def _():
        pltpu.sync_copy(zeros_hbm.at[pl.ds(0, TRASH)], acc.at[pl.ds(HALF, TRASH)])
    plsc.subcore_barrier()

    @pl.when(nblk > 0)
    def _():
        pltpu.async_copy(csrc_hbm.at[c, s, pl.ds(0, NB)], srcb.at[0], semi)
        pltpu.async_copy(cdst_hbm.at[c, s, pl.ds(0, NB)], dstb.at[0], semi)

    def blk_body(blk, carry):
        cur = lax.rem(blk, 2)
        nxt = 1 - cur
        pltpu.make_async_copy(csrc_hbm.at[c, s, pl.ds(blk * NB, NB)],
                              srcb.at[cur], semi).wait()
        pltpu.make_async_copy(cdst_hbm.at[c, s, pl.ds(blk * NB, NB)],
                              dstb.at[cur], semi).wait()

        @pl.when(blk + 1 < nblk)
        def _():
            pltpu.async_copy(csrc_hbm.at[c, s, pl.ds((blk + 1) * NB, NB)],
                             srcb.at[nxt], semi)
            pltpu.async_copy(cdst_hbm.at[c, s, pl.ds((blk + 1) * NB, NB)],
                             dstb.at[nxt], semi)

        pltpu.async_copy(h_hbm.at[srcb.at[cur, 0]], rows3.at[0], semg)
        pltpu.async_copy(h_hbm.at[srcb.at[cur, 1]], rows3.at[1], semg)
        for j in range(NB):
            pltpu.make_async_copy(h_hbm.at[srcb.at[cur, j]],
                                  rows3.at[j % 3], semg).wait()
            if j > 0:
                pltpu.make_async_copy(rows3.at[(j - 1) % 3],
                                      acc.at[dstb.at[cur, j - 1]], sems).wait()
            if j + 2 < NB:
                pltpu.async_copy(h_hbm.at[srcb.at[cur, j + 2]],
                                 rows3.at[(j + 2) % 3], semg)
            pltpu.async_copy(rows3.at[j % 3], acc.at[dstb.at[cur, j]],
                             sems, add=True)
        pltpu.make_async_copy(rows3.at[(NB - 1) % 3],
                              acc.at[dstb.at[cur, NB - 1]], sems).wait()
        return carry

    lax.fori_loop(0, nblk, blk_body, 0)
    plsc.subcore_barrier()

    def rbody(j, carry):
        i = s + j * NS
        @pl.when(i < NZC)
        def _():
            pltpu.sync_copy(acc.at[pl.ds(i * ZCH, ZCH)],
                            out_hbm.at[pl.ds(c * HALF + i * ZCH, ZCH)])
        return carry

    lax.fori_loop(0, NZC // NS + 1, rbody, 0)


B = 2000
GRID = N // B


def _row_spec(w):
    return pl.BlockSpec((B, w), lambda i: (i, 0))


def _full_spec(a, b):
    return pl.BlockSpec((a, b), lambda i: (0, 0))


def _dot(x, w):
    return jnp.dot(x, w, preferred_element_type=jnp.float32)


def _embed_body(prot, drug, dis, wp, bp, wd, bd, wq, bq, hp, hd, hq):
    hp[...] = _dot(prot[...], wp[...]) + bp[...]
    hd[...] = _dot(drug[...], wd[...]) + bd[...]
    hq[...] = _dot(dis[...], wq[...]) + bq[...]


def _embed(prot, drug, dis, wp, bp, wd, bd, wq, bq):
    return pl.pallas_call(
        _embed_body,
        grid=(GRID,),
        in_specs=[_row_spec(128), _row_spec(D), _row_spec(D),
                  _full_spec(128, D), _full_spec(1, D),
                  _full_spec(D, D), _full_spec(1, D),
                  _full_spec(D, D), _full_spec(1, D)],
        out_specs=[_row_spec(D), _row_spec(D), _row_spec(D)],
        out_shape=[jax.ShapeDtypeStruct((N, D), jnp.float32)] * 3,
    )(prot, drug, dis, wp, bp, wd, bd, wq, bq)


def _sage_body(h, agg, deg, ws, wn, b, o):
    mean = agg[...] / jnp.maximum(deg[...], 1.0)
    o[...] = _dot(h[...], ws[...]) + _dot(mean, wn[...]) + b[...]


def _sage_combine(h, agg, deg, ws, wn, b):
    return pl.pallas_call(
        _sage_body,
        grid=(GRID,),
        in_specs=[_row_spec(D), _row_spec(D), _row_spec(1),
                  _full_spec(D, D), _full_spec(D, D), _full_spec(1, D)],
        out_specs=_row_spec(D),
        out_shape=jax.ShapeDtypeStruct((N, D), jnp.float32),
    )(h, agg, deg, ws, wn, b)


def _gcnpre_body(h, dout, w, o):
    o[...] = _dot(h[...], w[...]) * lax.rsqrt(jnp.maximum(dout[...], 1.0))


def _gcn_pre(h, dout, w):
    return pl.pallas_call(
        _gcnpre_body,
        grid=(GRID,),
        in_specs=[_row_spec(D), _row_spec(1), _full_spec(D, D)],
        out_specs=_row_spec(D),
        out_shape=jax.ShapeDtypeStruct((N, D), jnp.float32),
    )(h, dout, w)


def _heads_body(agg, din, hdrug, hdis, bg,
                w_rna, b_rna, wd_a, wd_b, b_dti, w_pw, b_pw,
                wdd_a, wdd_b, b_dd, w2, b2, o):
    h = agg[...] * lax.rsqrt(jnp.maximum(din[...], 1.0)) + bg[...]
    hd = hdrug[...]
    rna_h = jax.nn.relu(_dot(h, w_rna[...]) + b_rna[...])
    dti_h = jax.nn.relu(_dot(hd, wd_a[...]) + _dot(h, wd_b[...]) + b_dti[...])
    pw_h = jax.nn.relu(_dot(h, w_pw[...]) + b_pw[...])
    dd_h = jax.nn.relu(_dot(hd, wdd_a[...]) + _dot(hdis[...], wdd_b[...]) + b_dd[...])
    hs = jnp.concatenate([rna_h, dti_h, pw_h, dd_h], axis=1)
    o[...] = jax.nn.sigmoid(_dot(hs, w2[...]) + b2[...])


def _heads(agg, din, hdrug, hdis, bg, wr, br, wda, wdb, bd, wp_, bp_,
           wdda, wddb, bdd, w2, b2):
    return pl.pallas_call(
        _heads_body,
        grid=(GRID,),
        in_specs=[_row_spec(D), _row_spec(1), _row_spec(D), _row_spec(D),
                  _full_spec(1, D),
                  _full_spec(D, D), _full_spec(1, D),
                  _full_spec(D, D), _full_spec(D, D), _full_spec(1, D),
                  _full_spec(D, D), _full_spec(1, D),
                  _full_spec(D, D), _full_spec(D, D), _full_spec(1, D),
                  _full_spec(4 * D, 4), _full_spec(1, 4)],
        out_specs=_row_spec(4),
        out_shape=jax.ShapeDtypeStruct((N, 4), jnp.float32),
    )(agg, din, hdrug, hdis, bg, wr, br, wda, wdb, bd, wp_, bp_,
      wdda, wddb, bdd, w2, b2)


def kernel(edge_index, drug_features, protein_features, cell_line_features,
           disease_features, params):
    src = edge_index[0].astype(jnp.int32)
    dst = edge_index[1].astype(jnp.int32)
    npad = E_PAD - E
    src_p = jnp.concatenate([src, jnp.full((npad,), N, jnp.int32)]).reshape(EROWS, CHUNK)
    dst_p = jnp.concatenate([dst, jnp.full((npad,), N, jnp.int32)]).reshape(EROWS, CHUNK)

    zeros2 = jnp.zeros((ZCH, D), jnp.float32)
    zeros1 = jnp.zeros((CHUNK,), jnp.float32)

    p = params
    b = lambda k: p[k].reshape(1, -1)

    csrc, cdst, cnt, deg_in, deg_out = _prep_kernel(src_p, dst_p, zeros1)
    csrc = csrc.reshape(NC, NS, NRMAX, CHUNK)
    cdst = cdst.reshape(NC, NS, NRMAX, CHUNK)
    deg_in = deg_in.reshape(N, 1)
    deg_out = deg_out.reshape(N, 1)

    h_prot, h_drug, h_dis = _embed(
        protein_features, drug_features, disease_features,
        p['W_prot'], b('b_prot'), p['W_drug'], b('b_drug'),
        p['W_dis'], b('b_dis'))

    h = h_prot
    for i in range(2):
        agg = _agg_kernel(csrc, cdst, cnt, h, zeros2)
        h = _sage_combine(h, agg, deg_in,
                          p['sage%d_Wself' % i], p['sage%d_Wneigh' % i],
                          b('sage%d_b' % i))

    hw = _gcn_pre(h, deg_out, p['W_gcn'])
    agg3 = _agg_kernel(csrc, cdst, cnt, hw, zeros2)

    w2 = jax.scipy.linalg.block_diag(p['rna_W2'], p['dti_W2'],
                                     p['pathway_W2'], p['dd_W2'])
    b2 = jnp.concatenate([p['rna_b2'], p['dti_b2'],
                          p['pathway_b2'], p['dd_b2']]).reshape(1, 4)

    return _heads(
        agg3, deg_in, h_drug, h_dis, b('b_gcn'),
        p['rna_W1'][:D] + p['rna_W1'][D:], b('rna_b1'),
        p['dti_W1'][:D], p['dti_W1'][D:], b('dti_b1'),
        p['pathway_W1'][:D] + p['pathway_W1'][D:], b('pathway_b1'),
        p['dd_W1'][:D], p['dd_W1'][D:], b('dd_b1'),
        w2, b2)


# speedup vs baseline: 1.0223x; 1.0223x over previous
import functools

import jax
import jax.numpy as jnp
from jax import lax
from jax.experimental import pallas as pl
from jax.experimental.pallas import tpu as pltpu
from jax.experimental.pallas import tpu_sc as plsc

N = 50000
E = 800000
D = 64

NC = 2
NS = 16
HALF = N // NC
TRASH = 16
ACC = HALF + TRASH

CHUNK = 128
NB = 8
BLKE = NB * CHUNK
ET = 50176
E_PAD = NS * ET
EROWS = E_PAD // CHUNK
TROWS = ET // CHUNK
NBLKP = TROWS // NB

TCAP = ET + BLKE
NRMAX = TCAP // CHUNK
NBMAX = TCAP // BLKE

NZFULL = 195
ZREM = ACC - NZFULL * CHUNK
RREM = HALF - NZFULL * CHUNK
NZITER = NZFULL // NS + 1

ZCH = 1000
NZC = HALF // ZCH

_mesh = plsc.VectorSubcoreMesh(core_axis_name="c", subcore_axis_name="s")
_sc_params = pltpu.CompilerParams(use_tc_tiling_on_sc=False)
_sc_params_nl = pltpu.CompilerParams(use_tc_tiling_on_sc=False,
                                     needs_layout_passes=False)


@functools.partial(
    pl.kernel,
    out_type=(jax.ShapeDtypeStruct((NC, NS, TCAP), jnp.int32),
              jax.ShapeDtypeStruct((NC, NS, TCAP), jnp.int32),
              jax.ShapeDtypeStruct((NC, NS, 16), jnp.int32),
              jax.ShapeDtypeStruct((N,), jnp.float32),
              jax.ShapeDtypeStruct((N,), jnp.float32)),
    mesh=_mesh,
    compiler_params=_sc_params_nl,
    scratch_types=[
        pltpu.VMEM((2, NB, CHUNK), jnp.int32),
        pltpu.VMEM((2, NB, CHUNK), jnp.int32),
        pltpu.VMEM((TCAP,), jnp.int32),
        pltpu.VMEM((TCAP,), jnp.int32),
        pltpu.VMEM((CHUNK,), jnp.int32),
        pltpu.VMEM((CHUNK,), jnp.int32),
        pltpu.VMEM((CHUNK,), jnp.float32),
        pltpu.VMEM((CHUNK,), jnp.float32),
        pltpu.VMEM((16,), jnp.int32),
        pltpu.VMEM_SHARED((ACC,), jnp.float32),
        pltpu.VMEM_SHARED((ACC,), jnp.float32),
        pltpu.SemaphoreType.DMA,
    ],
)
def _prep_kernel(src_hbm, dst_hbm, zeros1_hbm,
                 csrc_hbm, cdst_hbm, cnt_hbm, din_hbm, dout_hbm,
                 srcb, dstb, csrcv, cdstv, dlocb, slocb, ones, zb, cntv,
                 accin, accout, semi):
    c = lax.axis_index("c")
    s = lax.axis_index("s")
    off = c * HALF
    lane = lax.iota(jnp.int32, 16)

    for j in range(CHUNK // 16):
        ones[pl.ds(j * 16, 16)] = jnp.full((16,), 1.0, jnp.float32)
    pltpu.sync_copy(zeros1_hbm, zb)

    def zbody(j, carry):
        i = s + j * NS
        @pl.when(i < NZFULL)
        def _():
            pltpu.sync_copy(zb, accin.at[pl.ds(i * CHUNK, CHUNK)])
            pltpu.sync_copy(zb, accout.at[pl.ds(i * CHUNK, CHUNK)])
        return carry

    lax.fori_loop(0, NZITER, zbody, 0)
    @pl.when(s == NS - 1)
    def _():
        pltpu.sync_copy(zb.at[pl.ds(0, ZREM)], accin.at[pl.ds(NZFULL * CHUNK, ZREM)])
        pltpu.sync_copy(zb.at[pl.ds(0, ZREM)], accout.at[pl.ds(NZFULL * CHUNK, ZREM)])
    plsc.subcore_barrier()

    base0 = s * TROWS
    pltpu.async_copy(src_hbm.at[pl.ds(base0, NB)], srcb.at[0], semi)
    pltpu.async_copy(dst_hbm.at[pl.ds(base0, NB)], dstb.at[0], semi)

    def blk_body(blk, pos):
        cur = lax.rem(blk, 2)
        nxt = 1 - cur
        base = s * TROWS + blk * NB
        pltpu.make_async_copy(src_hbm.at[pl.ds(base, NB)], srcb.at[cur], semi).wait()
        pltpu.make_async_copy(dst_hbm.at[pl.ds(base, NB)], dstb.at[cur], semi).wait()

        @pl.when(blk + 1 < NBLKP)
        def _():
            nb_ = base + NB
            pltpu.async_copy(src_hbm.at[pl.ds(nb_, NB)], srcb.at[nxt], semi)
            pltpu.async_copy(dst_hbm.at[pl.ds(nb_, NB)], dstb.at[nxt], semi)

        one16 = jnp.full((16,), 1, jnp.int32)
        zero16 = jnp.zeros((16,), jnp.int32)

        def row_body(r, posv16):
            for j in range(CHUNK // 16):
                sv = srcb[cur, r, pl.ds(j * 16, 16)]
                dv = dstb[cur, r, pl.ds(j * 16, 16)]
                dl = dv - off
                dm = (dl >= 0) & (dl < HALF)
                dlocb[pl.ds(j * 16, 16)] = jnp.where(dm, dl, HALF + lane)
                sl = sv - off
                sm = (sl >= 0) & (sl < HALF)
                slocb[pl.ds(j * 16, 16)] = jnp.where(sm, sl, HALF + lane)
                dmi = jnp.where(dm, one16, zero16)
                inc = plsc.cumsum(dmi)
                idxv = posv16 + (inc - dmi)
                plsc.store_scatter(csrcv, [idxv], sv, mask=dm)
                plsc.store_scatter(cdstv, [idxv], dl, mask=dm)
                posv16 = posv16 + plsc.all_reduce_population_count(dm)
            pltpu.sync_copy(ones, accin.at[dlocb], add=True)
            pltpu.sync_copy(ones, accout.at[slocb], add=True)
            return posv16

        return lax.fori_loop(0, NB, row_body, pos)

    posv = lax.fori_loop(0, NBLKP, blk_body, jnp.zeros((16,), jnp.int32))

    lane = lax.iota(jnp.int32, 16)
    zero16 = jnp.zeros((16,), jnp.int32)
    for i in range(BLKE // 16):
        idxv = posv + (i * 16) + lane
        plsc.store_scatter(csrcv, [idxv], zero16)
        plsc.store_scatter(cdstv, [idxv], HALF + lane)
    cntv[pl.ds(0, 16)] = jnp.right_shift(posv + (BLKE - 1), 10)
    pltpu.sync_copy(cntv, cnt_hbm.at[c, s])
    pltpu.sync_copy(csrcv, csrc_hbm.at[c, s])
    pltpu.sync_copy(cdstv, cdst_hbm.at[c, s])
    plsc.subcore_barrier()

    def rbody(j, carry):
        i = s + j * NS
        @pl.when(i < NZFULL)
        def _():
            pltpu.sync_copy(accin.at[pl.ds(i * CHUNK, CHUNK)], zb)
            pltpu.sync_copy(zb, din_hbm.at[pl.ds(c * HALF + i * CHUNK, CHUNK)])
            pltpu.sync_copy(accout.at[pl.ds(i * CHUNK, CHUNK)], zb)
            pltpu.sync_copy(zb, dout_hbm.at[pl.ds(c * HALF + i * CHUNK, CHUNK)])
        return carry

    lax.fori_loop(0, NZITER, rbody, 0)
    @pl.when(s == NS - 1)
    def _():
        pltpu.sync_copy(accin.at[pl.ds(NZFULL * CHUNK, RREM)], zb.at[pl.ds(0, RREM)])
        pltpu.sync_copy(zb.at[pl.ds(0, RREM)],
                        din_hbm.at[pl.ds(c * HALF + NZFULL * CHUNK, RREM)])
        pltpu.sync_copy(accout.at[pl.ds(NZFULL * CHUNK, RREM)], zb.at[pl.ds(0, RREM)])
        pltpu.sync_copy(zb.at[pl.ds(0, RREM)],
                        dout_hbm.at[pl.ds(c * HALF + NZFULL * CHUNK, RREM)])


@functools.partial(
    pl.kernel,
    out_type=jax.ShapeDtypeStruct((N, D), jnp.float32),
    mesh=_mesh,
    compiler_params=_sc_params_nl,
    scratch_types=[
        pltpu.VMEM((2, NB, CHUNK), jnp.int32),
        pltpu.VMEM((2, NB, CHUNK), jnp.int32),
        pltpu.VMEM((3, CHUNK, D), jnp.float32),
        pltpu.VMEM((16,), jnp.int32),
        pltpu.VMEM_SHARED((ACC, D), jnp.float32),
        pltpu.SemaphoreType.DMA,
        pltpu.SemaphoreType.DMA,
        pltpu.SemaphoreType.DMA,
    ],
)
def _agg_kernel(csrc_hbm, cdst_hbm, cnt_hbm, h_hbm, zeros_hbm, out_hbm,
                srcb, dstb, rows3, cntv, acc, semi, semg, sems):
    zrows = rows3.at[0]
    c = lax.axis_index("c")
    s = lax.axis_index("s")

    pltpu.sync_copy(cnt_hbm.at[c, s], cntv)
    nblk = jnp.max(cntv[pl.ds(0, 16)])

    pltpu.sync_copy(zeros_hbm.at[pl.ds(0, CHUNK)], zrows)

    def zbody(j, carry):
        i = s + j * NS
        @pl.when(i < NZFULL)
        def _():
            pltpu.sync_copy(zrows, acc.at[pl.ds(i * CHUNK, CHUNK)])
        return carry

    lax.fori_loop(0, NZITER, zbody, 0)
    @pl.when(s == NS - 1)
    def _():
        pltpu.sync_copy(zrows.at[pl.ds(0, ZREM)], acc.at[pl.ds(NZFULL * CHUNK, ZREM)])
    plsc.subcore_barrier()

    @pl.when(nblk > 0)
    def _():
        pltpu.async_copy(csrc_hbm.at[c, s, pl.ds(0, NB)], srcb.at[0], semi)
        pltpu.async_copy(cdst_hbm.at[c, s, pl.ds(0, NB)], dstb.at[0], semi)

    def blk_body(blk, carry):
        cur = lax.rem(blk, 2)
        nxt = 1 - cur
        pltpu.make_async_copy(csrc_hbm.at[c, s, pl.ds(blk * NB, NB)],
                              srcb.at[cur], semi).wait()
        pltpu.make_async_copy(cdst_hbm.at[c, s, pl.ds(blk * NB, NB)],
                              dstb.at[cur], semi).wait()

        @pl.when(blk + 1 < nblk)
        def _():
            pltpu.async_copy(csrc_hbm.at[c, s, pl.ds((blk + 1) * NB, NB)],
                             srcb.at[nxt], semi)
            pltpu.async_copy(cdst_hbm.at[c, s, pl.ds((blk + 1) * NB, NB)],
                             dstb.at[nxt], semi)

        pltpu.async_copy(h_hbm.at[srcb.at[cur, 0]], rows3.at[0], semg)
        pltpu.async_copy(h_hbm.at[srcb.at[cur, 1]], rows3.at[1], semg)
        for j in range(NB):
            pltpu.make_async_copy(h_hbm.at[srcb.at[cur, j]],
                                  rows3.at[j % 3], semg).wait()
            if j > 0:
                pltpu.make_async_copy(rows3.at[(j - 1) % 3],
                                      acc.at[dstb.at[cur, j - 1]], sems).wait()
            if j + 2 < NB:
                pltpu.async_copy(h_hbm.at[srcb.at[cur, j + 2]],
                                 rows3.at[(j + 2) % 3], semg)
            pltpu.async_copy(rows3.at[j % 3], acc.at[dstb.at[cur, j]],
                             sems, add=True)
        pltpu.make_async_copy(rows3.at[(NB - 1) % 3],
                              acc.at[dstb.at[cur, NB - 1]], sems).wait()
        return carry

    lax.fori_loop(0, nblk, blk_body, 0)
    plsc.subcore_barrier()

    def rbody(j, carry):
        i = s + j * NS
        @pl.when(i < NZFULL)
        def _():
            pltpu.sync_copy(acc.at[pl.ds(i * CHUNK, CHUNK)], zrows)
            pltpu.sync_copy(zrows, out_hbm.at[pl.ds(c * HALF + i * CHUNK, CHUNK)])
        return carry

    lax.fori_loop(0, NZITER, rbody, 0)
    @pl.when(s == NS - 1)
    def _():
        pltpu.sync_copy(acc.at[pl.ds(NZFULL * CHUNK, RREM)], zrows.at[pl.ds(0, RREM)])
        pltpu.sync_copy(zrows.at[pl.ds(0, RREM)],
                        out_hbm.at[pl.ds(c * HALF + NZFULL * CHUNK, RREM)])


B = 2000
GRID = N // B


def _row_spec(w):
    return pl.BlockSpec((B, w), lambda i: (i, 0))


def _full_spec(a, b):
    return pl.BlockSpec((a, b), lambda i: (0, 0))


def _dot(x, w):
    return jnp.dot(x, w, preferred_element_type=jnp.float32)


def _embed_body(prot, drug, dis, wp, bp, wd, bd, wq, bq, hp, hd, hq):
    hp[...] = _dot(prot[...], wp[...]) + bp[...]
    hd[...] = _dot(drug[...], wd[...]) + bd[...]
    hq[...] = _dot(dis[...], wq[...]) + bq[...]


def _embed(prot, drug, dis, wp, bp, wd, bd, wq, bq):
    return pl.pallas_call(
        _embed_body,
        grid=(GRID,),
        in_specs=[_row_spec(128), _row_spec(D), _row_spec(D),
                  _full_spec(128, D), _full_spec(1, D),
                  _full_spec(D, D), _full_spec(1, D),
                  _full_spec(D, D), _full_spec(1, D)],
        out_specs=[_row_spec(D), _row_spec(D), _row_spec(D)],
        out_shape=[jax.ShapeDtypeStruct((N, D), jnp.float32)] * 3,
    )(prot, drug, dis, wp, bp, wd, bd, wq, bq)


def _sage_body(h, agg, deg, ws, wn, b, o):
    mean = agg[...] / jnp.maximum(deg[...], 1.0)
    o[...] = _dot(h[...], ws[...]) + _dot(mean, wn[...]) + b[...]


def _sage_combine(h, agg, deg, ws, wn, b):
    return pl.pallas_call(
        _sage_body,
        grid=(GRID,),
        in_specs=[_row_spec(D), _row_spec(D), _row_spec(1),
                  _full_spec(D, D), _full_spec(D, D), _full_spec(1, D)],
        out_specs=_row_spec(D),
        out_shape=jax.ShapeDtypeStruct((N, D), jnp.float32),
    )(h, agg, deg, ws, wn, b)


def _sagegcn_body(h, agg, deg, dout, ws, wn, b, wg, o):
    mean = agg[...] / jnp.maximum(deg[...], 1.0)
    h2 = _dot(h[...], ws[...]) + _dot(mean, wn[...]) + b[...]
    o[...] = _dot(h2, wg[...]) * lax.rsqrt(jnp.maximum(dout[...], 1.0))


def _sage_gcn(h, agg, deg, dout, ws, wn, b, wg):
    return pl.pallas_call(
        _sagegcn_body,
        grid=(GRID,),
        in_specs=[_row_spec(D), _row_spec(D), _row_spec(1), _row_spec(1),
                  _full_spec(D, D), _full_spec(D, D), _full_spec(1, D),
                  _full_spec(D, D)],
        out_specs=_row_spec(D),
        out_shape=jax.ShapeDtypeStruct((N, D), jnp.float32),
    )(h, agg, deg, dout, ws, wn, b, wg)


def _heads_body(agg, din, hdrug, hdis, bg,
                w_rna, b_rna, wd_a, wd_b, b_dti, w_pw, b_pw,
                wdd_a, wdd_b, b_dd, w2, b2, o):
    h = agg[...] * lax.rsqrt(jnp.maximum(din[...], 1.0)) + bg[...]
    hd = hdrug[...]
    rna_h = jax.nn.relu(_dot(h, w_rna[...]) + b_rna[...])
    dti_h = jax.nn.relu(_dot(hd, wd_a[...]) + _dot(h, wd_b[...]) + b_dti[...])
    pw_h = jax.nn.relu(_dot(h, w_pw[...]) + b_pw[...])
    dd_h = jax.nn.relu(_dot(hd, wdd_a[...]) + _dot(hdis[...], wdd_b[...]) + b_dd[...])
    hs = jnp.concatenate([rna_h, dti_h, pw_h, dd_h], axis=1)
    o[...] = jax.nn.sigmoid(_dot(hs, w2[...]) + b2[...])


def _heads(agg, din, hdrug, hdis, bg, wr, br, wda, wdb, bd, wp_, bp_,
           wdda, wddb, bdd, w2, b2):
    return pl.pallas_call(
        _heads_body,
        grid=(GRID,),
        in_specs=[_row_spec(D), _row_spec(1), _row_spec(D), _row_spec(D),
                  _full_spec(1, D),
                  _full_spec(D, D), _full_spec(1, D),
                  _full_spec(D, D), _full_spec(D, D), _full_spec(1, D),
                  _full_spec(D, D), _full_spec(1, D),
                  _full_spec(D, D), _full_spec(D, D), _full_spec(1, D),
                  _full_spec(4 * D, 4), _full_spec(1, 4)],
        out_specs=_row_spec(4),
        out_shape=jax.ShapeDtypeStruct((N, 4), jnp.float32),
    )(agg, din, hdrug, hdis, bg, wr, br, wda, wdb, bd, wp_, bp_,
      wdda, wddb, bdd, w2, b2)


def kernel(edge_index, drug_features, protein_features, cell_line_features,
           disease_features, params):
    src = edge_index[0].astype(jnp.int32)
    dst = edge_index[1].astype(jnp.int32)
    npad = E_PAD - E
    src_p = jnp.concatenate([src, jnp.full((npad,), N, jnp.int32)]).reshape(EROWS, CHUNK)
    dst_p = jnp.concatenate([dst, jnp.full((npad,), N, jnp.int32)]).reshape(EROWS, CHUNK)

    zeros2 = jnp.zeros((ZCH, D), jnp.float32)
    zeros1 = jnp.zeros((CHUNK,), jnp.float32)

    p = params
    b = lambda k: p[k].reshape(1, -1)

    csrc, cdst, cnt, deg_in, deg_out = _prep_kernel(src_p, dst_p, zeros1)
    csrc = csrc.reshape(NC, NS, NRMAX, CHUNK)
    cdst = cdst.reshape(NC, NS, NRMAX, CHUNK)
    deg_in = deg_in.reshape(N, 1)
    deg_out = deg_out.reshape(N, 1)

    h_prot, h_drug, h_dis = _embed(
        protein_features, drug_features, disease_features,
        p['W_prot'], b('b_prot'), p['W_drug'], b('b_drug'),
        p['W_dis'], b('b_dis'))

    agg1 = _agg_kernel(csrc, cdst, cnt, h_prot, zeros2)
    h1 = _sage_combine(h_prot, agg1, deg_in,
                       p['sage0_Wself'], p['sage0_Wneigh'], b('sage0_b'))
    agg2 = _agg_kernel(csrc, cdst, cnt, h1, zeros2)
    hw = _sage_gcn(h1, agg2, deg_in, deg_out,
                   p['sage1_Wself'], p['sage1_Wneigh'], b('sage1_b'),
                   p['W_gcn'])
    agg3 = _agg_kernel(csrc, cdst, cnt, hw, zeros2)

    w2 = jax.scipy.linalg.block_diag(p['rna_W2'], p['dti_W2'],
                                     p['pathway_W2'], p['dd_W2'])
    b2 = jnp.concatenate([p['rna_b2'], p['dti_b2'],
                          p['pathway_b2'], p['dd_b2']]).reshape(1, 4)

    return _heads(
        agg3, deg_in, h_drug, h_dis, b('b_gcn'),
        p['rna_W1'][:D] + p['rna_W1'][D:], b('rna_b1'),
        p['dti_W1'][:D], p['dti_W1'][D:], b('dti_b1'),
        p['pathway_W1'][:D] + p['pathway_W1'][D:], b('pathway_b1'),
        p['dd_W1'][:D], p['dd_W1'][D:], b('dd_b1'),
        w2, b2)
